# baseline (speedup 1.0000x reference)
import jax
import jax.numpy as jnp
from jax.experimental import pallas as pl

N = 50000
G = 64
H = 64
R = 2


def _copy_kernel(x_ref, o_ref):
    o_ref[...] = x_ref[...]


def _pl_copy(x):
    return pl.pallas_call(
        _copy_kernel,
        out_shape=jax.ShapeDtypeStruct(x.shape, x.dtype),
    )(x)


def _rel_msg(xs, etype, Wrel):
    m0 = xs @ Wrel[0]
    m1 = xs @ Wrel[1]
    return jnp.where((etype == 1)[:, None], m1, m0)


def _rgcn_edge(x, src, dst, etype, e, Wroot, Wrel, Wedge, b):
    m = _rel_msg(x[src], etype, Wrel) + e @ Wedge
    agg = jax.ops.segment_sum(m, dst, num_segments=N)
    deg = jax.ops.segment_sum(jnp.ones((dst.shape[0],), jnp.float32), dst, num_segments=N)
    return x @ Wroot + agg / jnp.maximum(deg, 1.0)[:, None] + b


def _rgcn_max(x, src, dst, etype, Wroot, Wrel, b):
    m = _rel_msg(x[src], etype, Wrel)
    seg = dst * R + etype
    agg = jax.ops.segment_max(m, seg, num_segments=N * R)
    agg = jnp.where(jnp.isfinite(agg), agg, 0.0).reshape(N, R, H).sum(axis=1)
    return x @ Wroot + agg + b


def _pool(h, batch):
    s = jax.ops.segment_sum(h, batch, num_segments=G)
    c = jax.ops.segment_sum(jnp.ones((h.shape[0],), jnp.float32), batch, num_segments=G)
    return s / jnp.maximum(c, 1.0)[:, None]


def kernel(x, action, edge_attr, n_enc1_W, n_enc1_b, e_enc1_W, e_enc1_b, conv1_Wroot, conv1_Wrel, conv1_Wedge, conv1_b, conv2_Wroot, conv2_Wrel, conv2_b, cls1_W, cls1_b, n_enc2_W, n_enc2_b, e_enc2_W, e_enc2_b, conv3_Wroot, conv3_Wrel, conv3_Wedge, conv3_b, conv4_Wroot, conv4_Wrel, conv4_b, cls2_W, cls2_b, edge_index, edge_type, batch):
    relu = jax.nn.relu
    src, dst = edge_index[0], edge_index[1]
    xa = jnp.concatenate([x, action], axis=1)
    n1 = relu(xa @ n_enc1_W + n_enc1_b)
    e1 = relu(edge_attr @ e_enc1_W + e_enc1_b)
    h1 = relu(_rgcn_edge(n1, src, dst, edge_type, e1, conv1_Wroot, conv1_Wrel, conv1_Wedge, conv1_b))
    h1 = relu(_rgcn_max(h1, src, dst, edge_type, conv2_Wroot, conv2_Wrel, conv2_b))
    h1 = h1 @ cls1_W + cls1_b
    out1 = _pool(h1, batch)
    n2 = relu(xa @ n_enc2_W + n_enc2_b)
    e2 = relu(edge_attr @ e_enc2_W + e_enc2_b)
    h2 = relu(_rgcn_edge(n2, src, dst, edge_type, e2, conv3_Wroot, conv3_Wrel, conv3_Wedge, conv3_b))
    h2 = relu(_rgcn_max(h2, src, dst, edge_type, conv4_Wroot, conv4_Wrel, conv4_b))
    h2 = h2 @ cls2_W + cls2_b
    out2 = _pool(h2, batch)
    return (_pl_copy(out1), _pl_copy(out2))


# speedup vs baseline: 2.0503x; 2.0503x over previous
import functools

import jax
import jax.numpy as jnp
from jax import lax
from jax.experimental import pallas as pl
from jax.experimental.pallas import tpu as pltpu
from jax.experimental.pallas import tpu_sc as plsc

N = 50000
E = 800000
G = 64
H = 64
R = 2

NC = 2
NS = 16
NW = NC * NS

NP = 50048
RPS = NP // NS
CH = 1000
CHE = 200
CHT = 200
EPW = E // NW
EPS = E // NS

_MESH = plsc.VectorSubcoreMesh(core_axis_name="c", subcore_axis_name="s")
_SC_PARAMS = pltpu.CompilerParams(use_tc_tiling_on_sc=False)
_f32 = jnp.float32


def _dot(a, b):
    return jnp.dot(a, b, preferred_element_type=jnp.float32)


def _gidx_kernel(src_ref, et_ref, o_ref):
    o_ref[...] = src_ref[...] * 2 + et_ref[...]


def _make_gidx(src, etype):
    src2 = src.reshape(6250, 128)
    et2 = etype.reshape(6250, 128)
    out = pl.pallas_call(
        _gidx_kernel,
        out_shape=jax.ShapeDtypeStruct((6250, 128), jnp.int32),
    )(src2, et2)
    return out.reshape(E)


def _enc_tables_kernel(xa_ref, wn_ref, bn_ref, wrel_ref, wroot_ref,
                       n_ref, tlo_ref, thi_ref, xr_ref):
    n = jnp.maximum(_dot(xa_ref[...], wn_ref[...]) + bn_ref[...], 0.0)
    n_ref[...] = n
    y0 = _dot(n, wrel_ref[0])
    y1 = _dot(n, wrel_ref[1])
    tlo_ref[:, 0, :] = y0[:, :32]
    tlo_ref[:, 1, :] = y1[:, :32]
    thi_ref[:, 0, :] = y0[:, 32:]
    thi_ref[:, 1, :] = y1[:, 32:]
    xr_ref[...] = _dot(n, wroot_ref[...])


def _enc_tables(xa, wn, bn, wrel, wroot):
    BN = 2000
    grid = (N // BN,)
    return pl.pallas_call(
        _enc_tables_kernel,
        grid=grid,
        in_specs=[
            pl.BlockSpec((BN, 4), lambda i: (i, 0)),
            pl.BlockSpec((4, H), lambda i: (0, 0)),
            pl.BlockSpec((1, H), lambda i: (0, 0)),
            pl.BlockSpec((R, H, H), lambda i: (0, 0, 0)),
            pl.BlockSpec((H, H), lambda i: (0, 0)),
        ],
        out_specs=[
            pl.BlockSpec((BN, H), lambda i: (i, 0)),
            pl.BlockSpec((BN, 2, 32), lambda i: (i, 0, 0)),
            pl.BlockSpec((BN, 2, 32), lambda i: (i, 0, 0)),
            pl.BlockSpec((BN, H), lambda i: (i, 0)),
        ],
        out_shape=[
            jax.ShapeDtypeStruct((N, H), _f32),
            jax.ShapeDtypeStruct((N, 2, 32), _f32),
            jax.ShapeDtypeStruct((N, 2, 32), _f32),
            jax.ShapeDtypeStruct((N, H), _f32),
        ],
    )(xa, wn, bn.reshape(1, H), wrel, wroot)


def _edge_enc_kernel(ea_ref, we_ref, be_ref, ee_ref):
    ea = ea_ref[...]
    we = we_ref[...]
    ee = ea[:, 0:1] * we[0:1, :] + ea[:, 1:2] * we[1:2, :] + be_ref[...]
    ee_ref[...] = jnp.maximum(ee, 0.0)


def _edge_enc(edge_attr, we, be):
    BE = 8000
    return pl.pallas_call(
        _edge_enc_kernel,
        grid=(E // BE,),
        in_specs=[
            pl.BlockSpec((BE, 2), lambda i: (i, 0)),
            pl.BlockSpec((2, 32), lambda i: (0, 0)),
            pl.BlockSpec((1, 32), lambda i: (0, 0)),
        ],
        out_specs=pl.BlockSpec((BE, 32), lambda i: (i, 0)),
        out_shape=jax.ShapeDtypeStruct((E, 32), _f32),
    )(edge_attr, we, be.reshape(1, 32))


def _combine_edge_kernel(xr_ref, alo_ref, ahi_ref, ae0_ref, ae1_ref,
                         d0_ref, d1_ref, wedge_ref, b_ref, wrel2_ref,
                         wroot2_ref, h_ref, zi_ref, xr2_ref):
    aggT = jnp.concatenate([alo_ref[...], ahi_ref[...]], axis=1)
    aggE = ae0_ref[...] + ae1_ref[...]
    eterm = _dot(aggE, wedge_ref[...])
    d = d0_ref[:, 0:1] + d1_ref[:, 0:1]
    rinv = 1.0 / jnp.maximum(d, 1.0)
    h = jnp.maximum(xr_ref[...] + (aggT + eterm) * rinv + b_ref[...], 0.0)
    h_ref[...] = h
    zi_ref[:, 0, :] = _dot(h, wrel2_ref[0])
    zi_ref[:, 1, :] = _dot(h, wrel2_ref[1])
    xr2_ref[...] = _dot(h, wroot2_ref[...])


def _combine_edge(xr, alo, ahi, ae0, ae1, d0, d1, wedge, b, wrel2, wroot2):
    BN = 2000
    return pl.pallas_call(
        _combine_edge_kernel,
        grid=(N // BN,),
        in_specs=[
            pl.BlockSpec((BN, H), lambda i: (i, 0)),
            pl.BlockSpec((BN, 32), lambda i: (i, 0)),
            pl.BlockSpec((BN, 32), lambda i: (i, 0)),
            pl.BlockSpec((BN, 32), lambda i: (i, 0)),
            pl.BlockSpec((BN, 32), lambda i: (i, 0)),
            pl.BlockSpec((BN, 8), lambda i: (i, 0)),
            pl.BlockSpec((BN, 8), lambda i: (i, 0)),
            pl.BlockSpec((32, H), lambda i: (0, 0)),
            pl.BlockSpec((1, H), lambda i: (0, 0)),
            pl.BlockSpec((R, H, H), lambda i: (0, 0, 0)),
            pl.BlockSpec((H, H), lambda i: (0, 0)),
        ],
        out_specs=[
            pl.BlockSpec((BN, H), lambda i: (i, 0)),
            pl.BlockSpec((BN, 2, H), lambda i: (i, 0, 0)),
            pl.BlockSpec((BN, H), lambda i: (i, 0)),
        ],
        out_shape=[
            jax.ShapeDtypeStruct((N, H), _f32),
            jax.ShapeDtypeStruct((N, 2, H), _f32),
            jax.ShapeDtypeStruct((N, H), _f32),
        ],
    )(xr, alo, ahi, ae0, ae1, d0, d1, wedge, b.reshape(1, H), wrel2, wroot2)


def _post_max_kernel(xr2_ref, mx_ref, b2_ref, clsw_ref, clsb_ref, batch_ref,
                     out_ref, s_acc, c_acc):
    i = pl.program_id(0)
    ng = pl.num_programs(0)
    m = mx_ref[...]
    m = jnp.where(m > -1e30, m, 0.0)
    h2 = jnp.maximum(xr2_ref[...] + m[:, 0, :] + m[:, 1, :] + b2_ref[...], 0.0)
    s = _dot(h2, clsw_ref[...]) + clsb_ref[0, 0]
    onehot = (batch_ref[...] == lax.broadcasted_iota(jnp.int32, (1, G), 1))
    onehot = onehot.astype(jnp.float32)
    ssum = jnp.sum(s * onehot, axis=0, keepdims=True)
    csum = jnp.sum(onehot, axis=0, keepdims=True)

    @pl.when(i == 0)
    def _():
        s_acc[...] = ssum
        c_acc[...] = csum

    @pl.when(i > 0)
    def _():
        s_acc[...] += ssum
        c_acc[...] += csum

    @pl.when(i == ng - 1)
    def _():
        out_ref[...] = s_acc[...] / jnp.maximum(c_acc[...], 1.0)


def _post_max(xr2, maxagg, b2, clsw, clsb, batch2):
    BN = 2000
    return pl.pallas_call(
        _post_max_kernel,
        grid=(N // BN,),
        in_specs=[
            pl.BlockSpec((BN, H), lambda i: (i, 0)),
            pl.BlockSpec((BN, 2, H), lambda i: (i, 0, 0)),
            pl.BlockSpec((1, H), lambda i: (0, 0)),
            pl.BlockSpec((H, 1), lambda i: (0, 0)),
            pl.BlockSpec((1, 1), lambda i: (0, 0)),
            pl.BlockSpec((BN, 1), lambda i: (i, 0)),
        ],
        out_specs=pl.BlockSpec((1, G), lambda i: (0, 0)),
        out_shape=jax.ShapeDtypeStruct((1, G), _f32),
        scratch_shapes=[
            pltpu.VMEM((1, G), _f32),
            pltpu.VMEM((1, G), _f32),
        ],
    )(xr2, maxagg, b2.reshape(1, H), clsw, clsb.reshape(1, 1), batch2)


def _deg_body(dst_hbm, ones_hbm, z8_hbm, deg_out, deg_s, idx_v, ones_v, sem):
    cid = lax.axis_index("c")
    sid = lax.axis_index("s")
    r0 = sid * RPS
    pltpu.sync_copy(ones_hbm, ones_v)
    pltpu.sync_copy(z8_hbm.at[pl.ds(r0, RPS), :], deg_s.at[pl.ds(r0, RPS), :])
    plsc.subcore_barrier()

    base = cid * (E // NC) + sid * EPW

    def chunk(t, carry):
        off = base + t * CH
        pltpu.sync_copy(dst_hbm.at[pl.ds(off, CH)], idx_v)
        pltpu.sync_copy(ones_v, deg_s.at[idx_v], add=True)
        return carry

    lax.fori_loop(0, EPW // CH, chunk, 0)
    plsc.subcore_barrier()

    @pl.when(cid == 0)
    def _():
        pltpu.sync_copy(deg_s.at[pl.ds(r0, RPS), :], deg_out.at[0, pl.ds(r0, RPS), :])

    @pl.when(cid == 1)
    def _():
        pltpu.sync_copy(deg_s.at[pl.ds(r0, RPS), :], deg_out.at[1, pl.ds(r0, RPS), :])


def _deg(dst):
    ones8 = jnp.ones((CH, 8), _f32)
    z8 = jnp.zeros((NP, 8), _f32)
    k = pl.kernel(
        _deg_body,
        out_type=jax.ShapeDtypeStruct((NC, NP, 8), _f32),
        mesh=_MESH,
        scratch_types=[
            pltpu.VMEM_SHARED((NP, 8), _f32),
            pltpu.VMEM((CH,), jnp.int32),
            pltpu.VMEM((CH, 8), _f32),
            pltpu.SemaphoreType.DMA,
        ],
        compiler_params=_SC_PARAMS,
    )
    return k(dst, ones8, z8)


def _ee_agg_body(ee_hbm, dst_hbm, z32_hbm, agg_out, acc_s, idx_v, rows_v, sem):
    cid = lax.axis_index("c")
    sid = lax.axis_index("s")
    r0 = sid * RPS
    pltpu.sync_copy(z32_hbm.at[pl.ds(r0, RPS), :], acc_s.at[pl.ds(r0, RPS), :])
    plsc.subcore_barrier()

    base = cid * (E // NC) + sid * EPW

    def chunk(t, carry):
        off = base + t * CHE
        pltpu.sync_copy(dst_hbm.at[pl.ds(off, CHE)], idx_v)
        pltpu.async_copy(ee_hbm.at[pl.ds(off, CHE), :], rows_v, sem).wait()
        pltpu.sync_copy(rows_v, acc_s.at[idx_v], add=True)
        return carry

    lax.fori_loop(0, EPW // CHE, chunk, 0)
    plsc.subcore_barrier()

    @pl.when(cid == 0)
    def _():
        pltpu.sync_copy(acc_s.at[pl.ds(r0, RPS), :], agg_out.at[0, pl.ds(r0, RPS), :])

    @pl.when(cid == 1)
    def _():
        pltpu.sync_copy(acc_s.at[pl.ds(r0, RPS), :], agg_out.at[1, pl.ds(r0, RPS), :])


def _ee_agg(ee, dst):
    z32 = jnp.zeros((NP, 32), _f32)
    k = pl.kernel(
        _ee_agg_body,
        out_type=jax.ShapeDtypeStruct((NC, NP, 32), _f32),
        mesh=_MESH,
        scratch_types=[
            pltpu.VMEM_SHARED((NP, 32), _f32),
            pltpu.VMEM((CHE,), jnp.int32),
            pltpu.VMEM((CHE, 32), _f32),
            pltpu.SemaphoreType.DMA,
        ],
        compiler_params=_SC_PARAMS,
    )
    return k(ee, dst, z32)


def _ti_agg_body(tlo_hbm, thi_hbm, gidx_hbm, dst_hbm, z32_hbm,
                 agg_out, acc_s, idx_v, didx_v, rows_v, sem):
    cid = lax.axis_index("c")
    sid = lax.axis_index("s")
    r0 = sid * RPS
    pltpu.sync_copy(z32_hbm.at[pl.ds(r0, RPS), :], acc_s.at[pl.ds(r0, RPS), :])
    plsc.subcore_barrier()

    base = sid * EPS

    def chunk(t, carry):
        off = base + t * CHT
        pltpu.sync_copy(gidx_hbm.at[pl.ds(off, CHT)], idx_v)
        pltpu.sync_copy(dst_hbm.at[pl.ds(off, CHT)], didx_v)

        @pl.when(cid == 0)
        def _():
            pltpu.async_copy(tlo_hbm.at[idx_v], rows_v, sem).wait()

        @pl.when(cid == 1)
        def _():
            pltpu.async_copy(thi_hbm.at[idx_v], rows_v, sem).wait()

        pltpu.sync_copy(rows_v, acc_s.at[didx_v], add=True)
        return carry

    lax.fori_loop(0, EPS // CHT, chunk, 0)
    plsc.subcore_barrier()

    @pl.when(cid == 0)
    def _():
        pltpu.sync_copy(acc_s.at[pl.ds(r0, RPS), :], agg_out.at[0, pl.ds(r0, RPS), :])

    @pl.when(cid == 1)
    def _():
        pltpu.sync_copy(acc_s.at[pl.ds(r0, RPS), :], agg_out.at[1, pl.ds(r0, RPS), :])


def _ti_agg(tlo, thi, gidx, dst):
    z32 = jnp.zeros((NP, 32), _f32)
    k = pl.kernel(
        _ti_agg_body,
        out_type=jax.ShapeDtypeStruct((NC, NP, 32), _f32),
        mesh=_MESH,
        scratch_types=[
            pltpu.VMEM_SHARED((NP, 32), _f32),
            pltpu.VMEM((CHT,), jnp.int32),
            pltpu.VMEM((CHT,), jnp.int32),
            pltpu.VMEM((CHT, 32), _f32),
            pltpu.SemaphoreType.DMA,
        ],
        compiler_params=_SC_PARAMS,
    )
    return k(tlo, thi, gidx, dst, z32)


def _tower(xa, edge_attr, src, dst, etype, gidx, batch2, deg_parts,
           wn, bn, we, be, wroot, wrel, wedge, b1,
           wroot2, wrel2, b2, clsw, clsb):
    _, tlo3, thi3, xroot = _enc_tables(xa, wn, bn, wrel, wroot)
    tlo = tlo3.reshape(2 * N, 32)
    thi = thi3.reshape(2 * N, 32)
    ee = _edge_enc(edge_attr, we, be)

    if deg_parts is None:
        deg2 = _deg(dst)
        deg_parts = (deg2[0, :N, :], deg2[1, :N, :])
    aggE2 = _ee_agg(ee, dst)
    aggT2 = _ti_agg(tlo, thi, gidx, dst)

    h1, zi, xroot2 = _combine_edge(
        xroot, aggT2[0, :N, :], aggT2[1, :N, :],
        aggE2[0, :N, :], aggE2[1, :N, :],
        deg_parts[0], deg_parts[1],
        wedge, b1, wrel2, wroot2)

    zi2 = zi.reshape(2 * N, H)
    m = zi2[gidx]
    seg = dst * R + etype
    maxagg = jax.ops.segment_max(m, seg, num_segments=N * R).reshape(N, R, H)

    pooled = _post_max(xroot2, maxagg, b2, clsw, clsb, batch2)
    return pooled.reshape(G, 1), deg_parts


def kernel(x, action, edge_attr, n_enc1_W, n_enc1_b, e_enc1_W, e_enc1_b, conv1_Wroot, conv1_Wrel, conv1_Wedge, conv1_b, conv2_Wroot, conv2_Wrel, conv2_b, cls1_W, cls1_b, n_enc2_W, n_enc2_b, e_enc2_W, e_enc2_b, conv3_Wroot, conv3_Wrel, conv3_Wedge, conv3_b, conv4_Wroot, conv4_Wrel, conv4_b, cls2_W, cls2_b, edge_index, edge_type, batch):
    src = edge_index[0]
    dst = edge_index[1]
    etype = edge_type.astype(jnp.int32)
    xa = jnp.concatenate([x, action], axis=1)
    batch2 = batch.astype(jnp.int32).reshape(N, 1)
    gidx = _make_gidx(src.astype(jnp.int32), etype)

    out1, deg_parts = _tower(
        xa, edge_attr, src, dst, etype, gidx, batch2, None,
        n_enc1_W, n_enc1_b, e_enc1_W, e_enc1_b, conv1_Wroot, conv1_Wrel,
        conv1_Wedge, conv1_b, conv2_Wroot, conv2_Wrel, conv2_b, cls1_W, cls1_b)
    out2, _ = _tower(
        xa, edge_attr, src, dst, etype, gidx, batch2, deg_parts,
        n_enc2_W, n_enc2_b, e_enc2_W, e_enc2_b, conv3_Wroot, conv3_Wrel,
        conv3_Wedge, conv3_b, conv4_Wroot, conv4_Wrel, conv4_b, cls2_W, cls2_b)
    return (out1, out2)
